# Initial kernel scaffold; baseline (speedup 1.0000x reference)
import jax
import jax.numpy as jnp
from jax.experimental import pallas as pl


def kernel(input, codes, codebooks, scales, bias):
    raise NotImplementedError("write your pallas kernel here")


# speedup vs baseline: 5.9073x; 5.9073x over previous
import functools

import jax
import jax.numpy as jnp
from jax import lax
from jax.experimental import pallas as pl
from jax.experimental.pallas import tpu as pltpu
from jax.experimental.pallas import tpu_sc as plsc

_IN_FEATURES = 4096
_OUT_FEATURES = 4096
_IN_GROUP = 8
_NUM_CB = 2
_CB_SIZE = 256
_NIG = _IN_FEATURES // _IN_GROUP

_NW = 32
_ROWS_PER_W = _OUT_FEATURES // _NW
_RCHUNK = 2
_NSTEPS = _ROWS_PER_W // _RCHUNK


def _dequant_body(codes_hbm, tab_hbm, w_hbm, cbuf, tab_v, obuf):
    wid = lax.axis_index("s") * 2 + lax.axis_index("c")
    row0 = wid * _ROWS_PER_W

    pltpu.sync_copy(tab_hbm, tab_v)

    lane = lax.iota(jnp.int32, 16)
    even = lane * 2
    odd = even + 1
    pos8 = lane * 8

    def step(s, carry):
        base = row0 + s * _RCHUNK
        pltpu.sync_copy(
            codes_hbm.at[pl.ds(base * _NIG * _NUM_CB, _RCHUNK * _NIG * _NUM_CB)],
            cbuf,
        )
        for r in range(_RCHUNK):
            for b in range(_NIG // 16):
                off = r * _NIG * _NUM_CB + b * 32
                c0 = plsc.load_gather(cbuf, [even + off])
                c1 = plsc.load_gather(cbuf, [odd + off])
                c0 = c0 * 8
                c1 = c1 * 8 + (_CB_SIZE * _IN_GROUP)
                for j in range(_IN_GROUP):
                    v0 = plsc.load_gather(tab_v, [c0 + j])
                    v1 = plsc.load_gather(tab_v, [c1 + j])
                    dst = pos8 + (r * _IN_FEATURES + b * 128 + j)
                    plsc.store_scatter(obuf, [dst], v0 + v1)
        pltpu.sync_copy(
            obuf, w_hbm.at[pl.ds(base * _IN_FEATURES, _RCHUNK * _IN_FEATURES)]
        )
        return carry

    lax.fori_loop(0, _NSTEPS, step, 0)


def _sc_dequant(codes_flat, tab):
    mesh = plsc.VectorSubcoreMesh(
        core_axis_name="c", subcore_axis_name="s", num_cores=2, num_subcores=16
    )
    return pl.kernel(
        _dequant_body,
        out_type=jax.ShapeDtypeStruct((_OUT_FEATURES * _IN_FEATURES,), jnp.float32),
        mesh=mesh,
        compiler_params=pltpu.CompilerParams(needs_layout_passes=False),
        scratch_types=[
            pltpu.VMEM((_RCHUNK * _NIG * _NUM_CB,), jnp.int32),
            pltpu.VMEM((_NUM_CB * _CB_SIZE * _IN_GROUP,), jnp.float32),
            pltpu.VMEM((_RCHUNK * _IN_FEATURES,), jnp.float32),
        ],
    )(codes_flat, tab)


def _gemm_kernel(x_ref, w_ref, s_ref, b_ref, o_ref, acc_ref, *, nk):
    k = pl.program_id(2)

    @pl.when(k == 0)
    def _():
        acc_ref[...] = jnp.zeros_like(acc_ref)

    xb = x_ref[...].astype(jnp.bfloat16)
    wb = w_ref[...].astype(jnp.bfloat16)
    acc_ref[...] += lax.dot_general(
        xb, wb, (((1,), (1,)), ((), ())), preferred_element_type=jnp.float32
    )

    @pl.when(k == nk - 1)
    def _():
        o_ref[...] = acc_ref[...] * s_ref[...] + b_ref[...]


def _tc_gemm(x, w, scales, bias, bm=1024, bn=1024, bk=512):
    m, k = x.shape
    n = w.shape[0]
    nk = k // bk
    grid = (m // bm, n // bn, nk)
    return pl.pallas_call(
        functools.partial(_gemm_kernel, nk=nk),
        grid=grid,
        in_specs=[
            pl.BlockSpec((bm, bk), lambda i, j, kk: (i, kk)),
            pl.BlockSpec((bn, bk), lambda i, j, kk: (j, kk)),
            pl.BlockSpec((1, bn), lambda i, j, kk: (0, j)),
            pl.BlockSpec((1, bn), lambda i, j, kk: (0, j)),
        ],
        out_specs=pl.BlockSpec((bm, bn), lambda i, j, kk: (i, j)),
        out_shape=jax.ShapeDtypeStruct((m, n), jnp.float32),
        scratch_shapes=[pltpu.VMEM((bm, bn), jnp.float32)],
        compiler_params=pltpu.CompilerParams(
            dimension_semantics=("parallel", "parallel", "arbitrary"),
        ),
    )(x, w, scales.reshape(1, n), bias.reshape(1, n))


def kernel(input, codes, codebooks, scales, bias):
    b, s, f = input.shape
    x = input.reshape(b * s, f)
    codes_flat = codes.reshape(-1)
    tab = codebooks.reshape(-1)
    w = _sc_dequant(codes_flat, tab).reshape(_OUT_FEATURES, _IN_FEATURES)
    out = _tc_gemm(x, w, scales.reshape(-1), bias)
    return out.reshape(b, s, _OUT_FEATURES)


# speedup vs baseline: 26.0244x; 4.4055x over previous
import functools

import jax
import jax.numpy as jnp
from jax import lax
from jax.experimental import pallas as pl
from jax.experimental.pallas import tpu as pltpu
from jax.experimental.pallas import tpu_sc as plsc

_IN_FEATURES = 4096
_OUT_FEATURES = 4096
_IN_GROUP = 8
_NUM_CB = 2
_CB_SIZE = 256
_NIG = _IN_FEATURES // _IN_GROUP

_NW = 32
_ROWS_PER_W = _OUT_FEATURES // _NW
_SLAB = 8
_NSTEPS = _ROWS_PER_W // _SLAB
_NBLK = _IN_FEATURES // 128


def _dequant_body(codes_hbm, tab_hbm, w_hbm, cbuf, tab_v, obuf):
    wid = lax.axis_index("s") * 2 + lax.axis_index("c")
    row0 = wid * _ROWS_PER_W

    pltpu.sync_copy(tab_hbm, tab_v)

    lane = lax.iota(jnp.int32, 16)
    zeros = lane * 0
    pos8 = lane * 8

    def step(s, carry):
        base = row0 + s * _SLAB
        pltpu.sync_copy(codes_hbm.at[pl.ds(base, _SLAB)], cbuf)

        def row(r, carry2):
            rsplat = zeros + r
            rcol = r * 128
            for b in range(_NBLK):
                cst = (b // 8) * 256 + (b % 8) * 16
                c0 = plsc.load_gather(cbuf, [rsplat, lane + cst])
                c1 = plsc.load_gather(cbuf, [rsplat, lane + (cst + 128)])
                c0 = c0 * 8
                c1 = c1 * 8 + (_CB_SIZE * _IN_GROUP)
                for j in range(_IN_GROUP):
                    v0 = plsc.load_gather(tab_v, [c0 + j])
                    v1 = plsc.load_gather(tab_v, [c1 + j])
                    dst = pos8 + (b * 1024 + rcol + j)
                    plsc.store_scatter(obuf, [dst], v0 + v1)
            return carry2

        lax.fori_loop(0, _SLAB, row, 0)
        pltpu.sync_copy(
            obuf, w_hbm.at[pl.ds(base * _IN_FEATURES, _SLAB * _IN_FEATURES)]
        )
        return carry

    lax.fori_loop(0, _NSTEPS, step, 0)


def _sc_dequant(codes, tab):
    mesh = plsc.VectorSubcoreMesh(
        core_axis_name="c", subcore_axis_name="s", num_cores=2, num_subcores=16
    )
    return pl.kernel(
        _dequant_body,
        out_type=jax.ShapeDtypeStruct((_OUT_FEATURES * _IN_FEATURES,), jnp.float32),
        mesh=mesh,
        scratch_types=[
            pltpu.VMEM((_SLAB, _NIG * _NUM_CB), jnp.int32),
            pltpu.VMEM((_NUM_CB * _CB_SIZE * _IN_GROUP,), jnp.float32),
            pltpu.VMEM((_SLAB * _IN_FEATURES,), jnp.float32),
        ],
        compiler_params=pltpu.CompilerParams(
            needs_layout_passes=False, use_tc_tiling_on_sc=False
        ),
    )(codes, tab)


def _gemm_kernel(x_ref, w_ref, s_ref, b_ref, o_ref, acc_ref, *, nk, bn, bk):
    k = pl.program_id(2)

    @pl.when(k == 0)
    def _():
        acc_ref[...] = jnp.zeros_like(acc_ref)

    xb = x_ref[...].astype(jnp.bfloat16)
    wb = jnp.swapaxes(w_ref[...], 1, 2).reshape(bn, bk).astype(jnp.bfloat16)
    acc_ref[...] += lax.dot_general(
        xb, wb, (((1,), (1,)), ((), ())), preferred_element_type=jnp.float32
    )

    @pl.when(k == nk - 1)
    def _():
        o_ref[...] = acc_ref[...] * s_ref[...] + b_ref[...]


def _tc_gemm(x, wt, scales, bias, bm=1024, bn=1024, bk=512):
    m, k = x.shape
    n = _OUT_FEATURES
    nk = k // bk
    grid = (m // bm, n // bn, nk)
    return pl.pallas_call(
        functools.partial(_gemm_kernel, nk=nk, bn=bn, bk=bk),
        grid=grid,
        in_specs=[
            pl.BlockSpec((bm, bk), lambda i, j, kk: (i, kk)),
            pl.BlockSpec((bn // 8, bk // 128, 8, 128), lambda i, j, kk: (j, kk, 0, 0)),
            pl.BlockSpec((1, bn), lambda i, j, kk: (0, j)),
            pl.BlockSpec((1, bn), lambda i, j, kk: (0, j)),
        ],
        out_specs=pl.BlockSpec((bm, bn), lambda i, j, kk: (i, j)),
        out_shape=jax.ShapeDtypeStruct((m, n), jnp.float32),
        scratch_shapes=[pltpu.VMEM((bm, bn), jnp.float32)],
        compiler_params=pltpu.CompilerParams(
            dimension_semantics=("parallel", "parallel", "arbitrary"),
        ),
    )(x, wt, scales.reshape(1, n), bias.reshape(1, n))


def kernel(input, codes, codebooks, scales, bias):
    b, s, f = input.shape
    x = input.reshape(b * s, f)
    tab = codebooks.reshape(-1)
    codes_sc = (
        codes.reshape(_OUT_FEATURES, _NIG // 128, 128, _NUM_CB)
        .transpose(0, 1, 3, 2)
        .reshape(_OUT_FEATURES, _NIG * _NUM_CB)
    )
    w_flat = _sc_dequant(codes_sc, tab)
    wt = w_flat.reshape(_OUT_FEATURES // 8, _NBLK, 8, 128)
    out = _tc_gemm(x, wt, scales.reshape(-1), bias)
    return out.reshape(b, s, _OUT_FEATURES)
